# baseline (speedup 1.0000x reference)
import functools

import jax
import jax.numpy as jnp
from jax import lax
from jax.experimental import pallas as pl
from jax.experimental.pallas import tpu as pltpu
from jax.experimental.pallas import tpu_sc as plsc

F32 = jnp.float32
_FULL = (1, 32, 384, 384, 1)


def _sc_gather_body(ml_ref, focus_ref, gp_ref, fvm, cvm, rows, out):
    pltpu.sync_copy(focus_ref, fvm)
    fv = fvm[...]
    z = fv[0]
    y = fv[1]
    x = fv[2]
    wid = lax.axis_index("s") * 2 + lax.axis_index("c")
    zz = wid // 2
    yhalf = wid % 2
    y64 = y + 64
    y64_al = (y64 // 8) * 8
    pltpu.sync_copy(ml_ref.at[z + 8, pl.ds(y64_al, 8), pl.ds(0, 128)], cvm)
    centre = plsc.load_gather(cvm, [jnp.full((16,), y64 - y64_al, jnp.int32),
                                    jnp.full((16,), x + 64, jnp.int32)])
    y_al = (y // 8) * 8
    dyy = y - y_al
    pltpu.sync_copy(
        ml_ref.at[z + zz, pl.ds(y_al + 64 * yhalf, 72), pl.ds(0, 256)],
        rows)
    lane = lax.broadcasted_iota(jnp.int32, (16,), 0)

    def row_step(r, _):
        for c in range(8):
            idx = lane + (x + 16 * c)
            v = plsc.load_gather(
                rows, [jnp.full((16,), dyy + r, jnp.int32), idx])
            out[r, pl.ds(16 * c, 16)] = jnp.where(v == centre, 1.0, 0.0)
        return 0

    lax.fori_loop(0, 64, row_step, 0)
    pltpu.sync_copy(out, gp_ref.at[pl.ds(wid * 64, 64), :])


def _gather(ml3, focus):
    mesh = plsc.VectorSubcoreMesh(core_axis_name="c", subcore_axis_name="s")
    k = pl.kernel(
        _sc_gather_body, mesh=mesh,
        compiler_params=pltpu.CompilerParams(needs_layout_passes=False),
        out_type=jax.ShapeDtypeStruct((2048, 128), F32),
        scratch_types=[pltpu.VMEM((16,), jnp.int32),
                       pltpu.VMEM((8, 128), jnp.int32),
                       pltpu.VMEM((72, 256), jnp.int32),
                       pltpu.VMEM((64, 128), F32)],
    )
    return k(ml3, jnp.pad(focus, (0, 13)))


def _l1_body(x_ref, w_ref, b_ref, o_ref):
    acc = jnp.zeros((1024, 2048), F32) + b_ref[...]
    for t in range(9):
        acc = acc + jnp.dot(x_ref[t], w_ref[t], preferred_element_type=F32)
    o_ref[...] = jnp.maximum(acc, 0.0)


def _l2_body(x_ref, w_ref, b_ref, o_ref):
    acc = jnp.zeros((8192, 64), F32) + b_ref[...]
    t = 0
    for jz in range(2):
        for jy in range(2):
            for jx in range(2):
                src = x_ref[jx, jz:jz + 8, jy:jy + 32, :, :]
                acc = acc + jnp.dot(src.reshape(8192, 256), w_ref[t],
                                    preferred_element_type=F32)
                t += 1
    o_ref[...] = jnp.maximum(acc, 0.0)


def _l3_body(x_ref, w_ref, b_ref, o_ref):
    acc = jnp.zeros((1024, 128), F32) + b_ref[...]
    t = 0
    for jz in range(2):
        for jy in range(2):
            for jx in range(2):
                src = x_ref[jx, jz:jz + 4, jy:jy + 16, :, :]
                acc = acc + jnp.dot(src.reshape(1024, 512), w_ref[t],
                                    preferred_element_type=F32)
                t += 1
    o_ref[...] = jnp.maximum(acc, 0.0)


def _conv_call(body, x, w, b, out_rows, out_ch):
    return pl.pallas_call(
        body,
        out_shape=jax.ShapeDtypeStruct((out_rows, out_ch), F32),
    )(x, w, b)


def _final_body(ret_ref, vis_ref, x4_ref, w4_ref, b4_ref, gp_ref,
                focus_ref, oret_ref, ovis_ref, rslab, vslab,
                sem1, sem2, sem3, sem4):
    z, y, x = focus_ref[0], focus_ref[1], focus_ref[2]
    c1 = pltpu.make_async_copy(
        ret_ref.at[pl.ds(z, 16), pl.ds(0, 144), :], rslab, sem1)
    c1.start()
    c2 = pltpu.make_async_copy(
        vis_ref.at[pl.ds(z, 16), pl.ds(0, 144), :], vslab, sem2)
    c2.start()
    acc = jnp.zeros((256, 1), F32) + b4_ref[...]
    t = 0
    for dz in range(3):
        for jy in range(2):
            for jx in range(2):
                src = x4_ref[jx, dz:dz + 4, jy:jy + 8, :, :]
                acc = acc + jnp.dot(src.reshape(256, 512), w4_ref[t],
                                    preferred_element_type=F32)
                t += 1
    p3 = (1.0 / (1.0 + jnp.exp(-acc))).reshape(4, 8, 8)
    yy = lax.broadcasted_iota(jnp.int32, (128, 8), 0)
    kk = lax.broadcasted_iota(jnp.int32, (128, 8), 1)
    ey = ((yy // 16) == kk).astype(F32)
    kk2 = lax.broadcasted_iota(jnp.int32, (8, 128), 0)
    xx = lax.broadcasted_iota(jnp.int32, (8, 128), 1)
    ex = (kk2 == (xx // 16)).astype(F32)
    ups = []
    for zc in range(4):
        a = jnp.dot(ey, p3[zc], preferred_element_type=F32)
        ups.append(jnp.dot(a, ex, preferred_element_type=F32)
                   .reshape(1, 128, 128))
    up4 = jnp.concatenate(ups, axis=0)
    up = jnp.broadcast_to(up4[:, None], (4, 4, 128, 128)).reshape(16, 128, 128)
    c2.wait()
    i0 = lax.broadcasted_iota(jnp.int32, (144, 384), 0)
    i1 = lax.broadcasted_iota(jnp.int32, (144, 384), 1)
    vmask = ((i0 == y) & (i1 == x)).astype(F32)
    v0 = jnp.sum(vslab[0:1, :, :].reshape(144, 384).astype(F32) * vmask)
    dof = jnp.where(v0 <= 3.5, 1.0, 0.0).reshape(1, 1, 1)
    gp = gp_ref[...].reshape(16, 128, 128)
    contrib = up * gp * dof
    gpd = gp * dof
    rk = lax.broadcasted_iota(jnp.int32, (144, 128), 0)
    cj = lax.broadcasted_iota(jnp.int32, (144, 128), 1)
    syt = (rk == y + cj).astype(F32)
    jj = lax.broadcasted_iota(jnp.int32, (128, 384), 0)
    kk3 = lax.broadcasted_iota(jnp.int32, (128, 384), 1)
    sxt = (kk3 == jj + x).astype(F32)
    rparts, vparts = [], []
    for zz in range(16):
        a = jnp.dot(syt, contrib[zz], preferred_element_type=F32)
        rparts.append(jnp.dot(a, sxt, preferred_element_type=F32)
                      .reshape(1, 144, 384))
        b = jnp.dot(syt, gpd[zz], preferred_element_type=F32)
        vparts.append(jnp.dot(b, sxt, preferred_element_type=F32)
                      .reshape(1, 144, 384))
    rc = jnp.concatenate(rparts, axis=0)
    vc = jnp.concatenate(vparts, axis=0)
    c1.wait()
    rslab[...] = jnp.maximum(rslab[...], rc)
    vslab[...] = vslab[...] + vc.astype(jnp.int32)
    co1 = pltpu.make_async_copy(
        rslab, oret_ref.at[pl.ds(z, 16), pl.ds(0, 144), :], sem3)
    co1.start()
    co2 = pltpu.make_async_copy(
        vslab, ovis_ref.at[pl.ds(z, 16), pl.ds(0, 144), :], sem4)
    co2.start()
    co1.wait()
    co2.wait()


def _final(ret3, vis3, x4, w4e, b4e, gp2, focus):
    return pl.pallas_call(
        _final_body,
        in_specs=[pl.BlockSpec(memory_space=pl.ANY),
                  pl.BlockSpec(memory_space=pl.ANY),
                  pl.BlockSpec(memory_space=pltpu.VMEM),
                  pl.BlockSpec(memory_space=pltpu.VMEM),
                  pl.BlockSpec(memory_space=pltpu.VMEM),
                  pl.BlockSpec(memory_space=pltpu.VMEM),
                  pl.BlockSpec(memory_space=pltpu.SMEM)],
        out_specs=[pl.BlockSpec(memory_space=pl.ANY),
                   pl.BlockSpec(memory_space=pl.ANY)],
        out_shape=[jax.ShapeDtypeStruct((32, 384, 384), F32),
                   jax.ShapeDtypeStruct((32, 384, 384), jnp.int32)],
        input_output_aliases={0: 0, 1: 1},
        scratch_shapes=[pltpu.VMEM((16, 144, 384), F32),
                        pltpu.VMEM((16, 144, 384), jnp.int32),
                        pltpu.SemaphoreType.DMA, pltpu.SemaphoreType.DMA,
                        pltpu.SemaphoreType.DMA, pltpu.SemaphoreType.DMA],
    )(ret3, vis3, x4, w4e, b4e, gp2, focus)


def _s2d(a, f):
    d, h, w, c = a.shape
    fz, fy, fx = f
    a = a.reshape(d // fz, fz, h // fy, fy, w // fx, fx, c)
    a = a.transpose(0, 2, 4, 1, 3, 5, 6)
    return a.reshape(d // fz, h // fy, w // fx, fz * fy * fx * c)


def _xpair(a, wout):
    return jnp.stack([a[:, :, 0:wout, :], a[:, :, 1:wout + 1, :]], axis=0)


def kernel(ret, machine_labels, visited, focus, W1, b1, W2, b2, W3, b3, W4, b4):
    ml3 = machine_labels.reshape(32, 384, 384)
    ret3 = ret.reshape(32, 384, 384)
    vis3 = visited.reshape(32, 384, 384)
    focus = focus.astype(jnp.int32)

    gp2 = _gather(ml3, focus)

    iox = jnp.arange(64)
    ixx = jnp.arange(128)
    idx = jnp.arange(3)
    xsel = (ixx[None, :, None] == 2 * iox[None, None, :]
            + idx[:, None, None]).astype(F32)
    w1e = jnp.einsum('axo,zyac->zyxoc', xsel,
                     W1[:, :, :, 0, :]).reshape(9, 128, 2048)
    b1e = jnp.tile(b1, 64).reshape(1, 2048)
    w2 = jnp.pad(W2, ((0, 1), (0, 1), (0, 1), (0, 0), (0, 0)))
    w2e = (w2.reshape(2, 2, 2, 2, 2, 2, 32, 64)
           .transpose(0, 2, 4, 1, 3, 5, 6, 7).reshape(8, 256, 64))
    w3 = jnp.pad(W3, ((0, 1), (0, 1), (0, 1), (0, 0), (0, 0)))
    w3e = (w3.reshape(2, 2, 2, 2, 2, 2, 64, 128)
           .transpose(0, 2, 4, 1, 3, 5, 6, 7).reshape(8, 512, 128))
    w4 = jnp.pad(W4[:, :, :, :, 0], ((0, 0), (0, 1), (0, 1), (0, 0)))
    w4e = (w4.reshape(3, 2, 2, 2, 2, 128).transpose(0, 1, 3, 2, 4, 5)
           .reshape(12, 512, 1))

    gpz = jnp.pad(gp2.reshape(16, 128, 128), ((1, 1), (0, 2), (0, 0)))
    a1 = jnp.stack([gpz[dz:dz + 16, dy:dy + 128:2, :]
                    for dz in range(3) for dy in range(3)], axis=0)
    out1 = _conv_call(_l1_body, a1.reshape(9, 1024, 128), w1e, b1e,
                      1024, 2048)

    x2 = _s2d(jnp.pad(out1.reshape(16, 64, 64, 32),
                      ((0, 2), (0, 2), (0, 2), (0, 0))), (2, 2, 2))
    out2 = _conv_call(_l2_body, _xpair(x2, 32), w2e, b2.reshape(1, 64),
                      8192, 64)

    x3 = _s2d(jnp.pad(out2.reshape(8, 32, 32, 64),
                      ((0, 2), (0, 2), (0, 2), (0, 0))), (2, 2, 2))
    out3 = _conv_call(_l3_body, _xpair(x3, 16), w3e, b3.reshape(1, 128),
                      1024, 128)

    x4 = _s2d(jnp.pad(out3.reshape(4, 16, 16, 128),
                      ((1, 1), (0, 2), (0, 2), (0, 0))), (1, 2, 2))

    ret_o, vis_o = _final(ret3, vis3, _xpair(x4, 8), w4e, b4.reshape(1, 1),
                          gp2, focus)
    return ret_o.reshape(_FULL), vis_o.reshape(_FULL)
